# Initial kernel scaffold; baseline (speedup 1.0000x reference)
import jax
import jax.numpy as jnp
from jax.experimental import pallas as pl


def kernel(u_proj, ps_proj, pf_proj, pb_proj, edge_index, fc_ln_g, fc_ln_b, fc_W, fc_b, ln_p_g, ln_p_b, W_p, att_src_p, att_dst_p, res_p, bias_p, ln_u_g, ln_u_b, W_u, att_src_u, att_dst_u, res_u, bias_u):
    raise NotImplementedError("write your pallas kernel here")


# speedup vs baseline: 14.2526x; 14.2526x over previous
import functools

import jax
import jax.numpy as jnp
from jax.experimental import pallas as pl
from jax.experimental.pallas import tpu as pltpu

HID = 64
NH = 8
CH = HID // NH
N = 50000
NBLK = 400


def _ln(x, g, b, eps=1e-5):
    mu = x.mean(-1, keepdims=True)
    var = ((x - mu) ** 2).mean(-1, keepdims=True)
    return (x - mu) * jax.lax.rsqrt(var + eps) * g + b


def _fc_body(ps_ref, pf_ref, pb_ref, g_ref, b_ref, w_ref, bias_ref, o_ref):
    cat = jnp.concatenate([ps_ref[...], pf_ref[...], pb_ref[...]], axis=-1)
    h = _ln(cat, g_ref[...], b_ref[...])
    o_ref[...] = h @ w_ref[...] + bias_ref[...]


def _fc_stage(ps, pf, pb, g, b, w, bias):
    grid = (N // NBLK,)
    blk = lambda r, c: pl.BlockSpec((NBLK, c), lambda i: (i, 0))
    full = lambda shape: pl.BlockSpec(shape, lambda i: tuple(0 for _ in shape))
    return pl.pallas_call(
        _fc_body,
        grid=grid,
        in_specs=[
            blk(NBLK, HID), blk(NBLK, HID), blk(NBLK, HID),
            full((3 * HID,)), full((3 * HID,)), full((3 * HID, HID)),
            full((HID,)),
        ],
        out_specs=pl.BlockSpec((NBLK, HID), lambda i: (i, 0)),
        out_shape=jax.ShapeDtypeStruct((N, HID), jnp.float32),
    )(ps, pf, pb, g, b, w, bias)


def _gat_pre_body(x_ref, lg_ref, lb_ref, w_ref, asrc_ref, adst_ref, rw_ref,
                  xw_ref, as_ref, ad_ref, res_ref):
    xn = _ln(x_ref[...], lg_ref[...], lb_ref[...])
    xw = xn @ w_ref[...]
    xw_ref[...] = xw
    x3 = xw.reshape(NBLK, NH, CH)
    as_ref[...] = (x3 * asrc_ref[...][None]).sum(-1)
    ad_ref[...] = (x3 * adst_ref[...][None]).sum(-1)
    res_ref[...] = xn @ rw_ref[...]


def _gat_pre(x, lg, lb, w, a_src, a_dst, res_w):
    grid = (N // NBLK,)
    blk = lambda c: pl.BlockSpec((NBLK, c), lambda i: (i, 0))
    full = lambda shape: pl.BlockSpec(shape, lambda i: tuple(0 for _ in shape))
    return pl.pallas_call(
        _gat_pre_body,
        grid=grid,
        in_specs=[
            blk(HID), full((HID,)), full((HID,)), full((HID, HID)),
            full((NH, CH)), full((NH, CH)), full((HID, HID)),
        ],
        out_specs=(blk(HID), blk(NH), blk(NH), blk(HID)),
        out_shape=(
            jax.ShapeDtypeStruct((N, HID), jnp.float32),
            jax.ShapeDtypeStruct((N, NH), jnp.float32),
            jax.ShapeDtypeStruct((N, NH), jnp.float32),
            jax.ShapeDtypeStruct((N, HID), jnp.float32),
        ),
    )(x, lg, lb, w, a_src, a_dst, res_w)


def _gat_layer(x, src, dst, lg, lb, w, a_src, a_dst, res_w, bias):
    xw, a_s, a_d, res = _gat_pre(x, lg, lb, w, a_src, a_dst, res_w)
    alpha = jax.nn.leaky_relu(a_s[src] + a_d[dst], 0.2)
    ex = jnp.exp(alpha)
    den = jax.ops.segment_sum(ex, dst, num_segments=N)
    acc = jax.ops.segment_sum(
        xw[src] * ex[:, :, None].reshape(-1, NH, 1).repeat(CH, -1).reshape(-1, HID),
        dst, num_segments=N)
    ex_self = jnp.exp(jax.nn.leaky_relu(a_s + a_d, 0.2))
    den = den + ex_self
    acc = acc + xw * jnp.repeat(ex_self, CH, axis=-1)
    out = acc / jnp.repeat(den, CH, axis=-1)
    return out + res + bias


def kernel(u_proj, ps_proj, pf_proj, pb_proj, edge_index, fc_ln_g, fc_ln_b,
           fc_W, fc_b, ln_p_g, ln_p_b, W_p, att_src_p, att_dst_p, res_p,
           bias_p, ln_u_g, ln_u_b, W_u, att_src_u, att_dst_u, res_u, bias_u):
    src = edge_index[0].astype(jnp.int32)
    dst = edge_index[1].astype(jnp.int32)
    p = _fc_stage(ps_proj, pf_proj, pb_proj, fc_ln_g, fc_ln_b, fc_W, fc_b)
    u1 = _gat_layer(p, src, dst, ln_p_g, ln_p_b, W_p, att_src_p, att_dst_p,
                    res_p, bias_p)
    out = _gat_layer(u1, src, dst, ln_u_g, ln_u_b, W_u, att_src_u, att_dst_u,
                     res_u, bias_u)
    return out.reshape(u_proj.shape)


# speedup vs baseline: 73.4894x; 5.1562x over previous
import functools

import jax
import jax.numpy as jnp
from jax import lax
from jax.experimental import pallas as pl
from jax.experimental.pallas import tpu as pltpu
from jax.experimental.pallas import tpu_sc as plsc

HID = 64
NH = 8
CH = HID // NH
N = 50000
E = 800000
NBLK = 400

HALF = 25088
NPAD = 2 * HALF
SENT = HALF
ACCR = HALF + 8
NSUB = 16
EPT = E // NSUB
CHUNK = 1000
NCHUNK = EPT // CHUNK
G = 128
CAPC = 1024
OUTR = HALF // NSUB


def _ln(x, g, b, eps=1e-5):
    mu = x.mean(-1, keepdims=True)
    var = ((x - mu) ** 2).mean(-1, keepdims=True)
    return (x - mu) * lax.rsqrt(var + eps) * g + b


def _fc_body(ps_ref, pf_ref, pb_ref, g_ref, b_ref, w_ref, bias_ref, o_ref):
    cat = jnp.concatenate([ps_ref[...], pf_ref[...], pb_ref[...]], axis=-1)
    h = _ln(cat, g_ref[...], b_ref[...])
    o_ref[...] = h @ w_ref[...] + bias_ref[...]


def _fc_stage(ps, pf, pb, g, b, w, bias):
    blk = lambda c: pl.BlockSpec((NBLK, c), lambda i: (i, 0))
    full = lambda shape: pl.BlockSpec(shape, lambda i: tuple(0 for _ in shape))
    return pl.pallas_call(
        _fc_body,
        grid=(N // NBLK,),
        in_specs=[blk(HID), blk(HID), blk(HID), full((3 * HID,)),
                  full((3 * HID,)), full((3 * HID, HID)), full((HID,))],
        out_specs=pl.BlockSpec((NBLK, HID), lambda i: (i, 0)),
        out_shape=jax.ShapeDtypeStruct((N, HID), jnp.float32),
    )(ps, pf, pb, g, b, w, bias)


def _gat_pre_body(x_ref, lg_ref, lb_ref, w_ref, asrc_ref, adst_ref, rw_ref,
                  xw_ref, asad_ref, res_ref):
    xn = _ln(x_ref[...], lg_ref[...], lb_ref[...])
    xw = xn @ w_ref[...]
    xw_ref[...] = xw
    x3 = xw.reshape(NBLK, NH, CH)
    a_s = (x3 * asrc_ref[...][None]).sum(-1)
    a_d = (x3 * adst_ref[...][None]).sum(-1)
    asad_ref[...] = jnp.concatenate([a_s, a_d], axis=-1)
    res_ref[...] = xn @ rw_ref[...]


def _gat_pre(x, lg, lb, w, a_src, a_dst, res_w):
    blk = lambda c: pl.BlockSpec((NBLK, c), lambda i: (i, 0))
    full = lambda shape: pl.BlockSpec(shape, lambda i: tuple(0 for _ in shape))
    return pl.pallas_call(
        _gat_pre_body,
        grid=(N // NBLK,),
        in_specs=[blk(HID), full((HID,)), full((HID,)), full((HID, HID)),
                  full((NH, CH)), full((NH, CH)), full((HID, HID))],
        out_specs=(blk(HID), blk(2 * NH), blk(HID)),
        out_shape=(jax.ShapeDtypeStruct((N, HID), jnp.float32),
                   jax.ShapeDtypeStruct((N, 2 * NH), jnp.float32),
                   jax.ShapeDtypeStruct((N, HID), jnp.float32)),
    )(x, lg, lb, w, a_src, a_dst, res_w)


def _edge_body(src_hbm, dst_hbm, xw_hbm, asad_hbm, z64_hbm, z8_hbm,
               acc_hbm, den_hbm,
               acc_sh, den_sh, sbuf, dbuf, csrc, cdl, slb, gdb, dlb,
               xwr, srows, drows, exb, sem):
    c = lax.axis_index("c")
    s = lax.axis_index("s")
    base = c * HALF
    i32 = jnp.int32
    iota = lax.broadcasted_iota(i32, (16,), 0)
    lane8 = iota & 7
    half_i = iota >> 3

    r0 = s * OUTR
    pltpu.sync_copy(z64_hbm, acc_sh.at[pl.ds(r0, OUTR)])
    pltpu.sync_copy(z8_hbm, den_sh.at[pl.ds(r0, OUTR)])

    @pl.when(s == NSUB - 1)
    def _zero_sentinel():
        pltpu.sync_copy(z64_hbm.at[pl.ds(0, 8)], acc_sh.at[pl.ds(HALF, 8)])
        pltpu.sync_copy(z8_hbm.at[pl.ds(0, 8)], den_sh.at[pl.ds(HALF, 8)])

    def _pre_src(i, carry):
        csrc[pl.ds(i * 16, 16)] = jnp.zeros((16,), i32)
        return carry
    lax.fori_loop(0, CAPC // 16, _pre_src, 0)

    plsc.subcore_barrier()

    def _chunk(k, carry):
        e0 = s * EPT + k * CHUNK
        pltpu.sync_copy(src_hbm.at[pl.ds(e0, CHUNK)], sbuf.at[pl.ds(0, CHUNK)])
        pltpu.sync_copy(dst_hbm.at[pl.ds(e0, CHUNK)], dbuf.at[pl.ds(0, CHUNK)])

        def _pre(i, carry2):
            cdl[pl.ds(i * 16, 16)] = jnp.full((16,), SENT, i32)
            return carry2
        lax.fori_loop(0, CAPC // 16, _pre, 0)

        def _filt(i, cnt):
            d = dbuf[pl.ds(i * 16, 16)]
            dl = d - base
            m = (dl >= 0) & (dl < HALF) & (i * 16 + iota < CHUNK)
            sv = sbuf[pl.ds(i * 16, 16)]
            csum = plsc.cumsum(m.astype(i32))
            pos = cnt + csum - 1
            plsc.store_scatter(cdl, [pos], dl, mask=m)
            plsc.store_scatter(csrc, [pos], sv, mask=m)
            return cnt + jnp.max(csum)
        cnt = lax.fori_loop(0, (CHUNK + 15) // 16, _filt, i32(0))

        nb = (cnt + (G - 1)) >> 7

        def _batch(b, carry3):
            def _cp(j, carry4):
                dv = cdl[pl.ds(b * G + j * 16, 16)]
                dlb[pl.ds(j * 16, 16)] = dv
                gdb[pl.ds(j * 16, 16)] = dv + base
                slb[pl.ds(j * 16, 16)] = csrc[pl.ds(b * G + j * 16, 16)]
                return carry4
            lax.fori_loop(0, G // 16, _cp, 0)

            cp1 = pltpu.async_copy(xw_hbm.at[slb], xwr, sem)
            cp2 = pltpu.async_copy(asad_hbm.at[slb], srows, sem)
            cp3 = pltpu.async_copy(asad_hbm.at[gdb], drows, sem)
            cp1.wait()
            cp2.wait()
            cp3.wait()

            def _pair(j, carry5):
                r2 = 2 * j + half_i
                a_s2 = plsc.load_gather(srows, [r2, lane8])
                a_d2 = plsc.load_gather(drows, [r2, lane8 + 8])
                a = a_s2 + a_d2
                a = jnp.where(a >= 0, a, 0.2 * a)
                e2 = jnp.exp(a)
                plsc.store_scatter(exb, [r2, lane8], e2)
                for q in range(4):
                    exa = plsc.load_gather(exb, [jnp.full((16,), 2 * j, i32),
                                                 2 * q + half_i])
                    exbv = plsc.load_gather(exb, [jnp.full((16,), 2 * j + 1, i32),
                                                  2 * q + half_i])
                    xa = xwr[2 * j, pl.ds(q * 16, 16)]
                    xb = xwr[2 * j + 1, pl.ds(q * 16, 16)]
                    xwr[2 * j, pl.ds(q * 16, 16)] = xa * exa
                    xwr[2 * j + 1, pl.ds(q * 16, 16)] = xb * exbv
                return carry5
            lax.fori_loop(0, G // 2, _pair, 0)

            pltpu.sync_copy(xwr, acc_sh.at[dlb], add=True)
            pltpu.sync_copy(exb, den_sh.at[dlb], add=True)
            return carry3
        lax.fori_loop(0, nb, _batch, 0)
        return carry
    lax.fori_loop(0, NCHUNK, _chunk, 0)

    plsc.subcore_barrier()

    pltpu.sync_copy(acc_sh.at[pl.ds(r0, OUTR)],
                    acc_hbm.at[pl.ds(base + r0, OUTR)])
    pltpu.sync_copy(den_sh.at[pl.ds(r0, OUTR)],
                    den_hbm.at[pl.ds(base + r0, OUTR)])


def _edge_call(src, dst, xw, asad, z64, z8):
    f32 = jnp.float32
    mesh = plsc.VectorSubcoreMesh(core_axis_name="c", subcore_axis_name="s")
    return pl.kernel(
        _edge_body,
        (jax.ShapeDtypeStruct((NPAD, HID), f32),
         jax.ShapeDtypeStruct((NPAD, NH), f32)),
        mesh=mesh,
        compiler_params=pltpu.CompilerParams(needs_layout_passes=False,
                                             use_tc_tiling_on_sc=False),
        scratch_types=[
            pltpu.VMEM_SHARED((ACCR, HID), f32),
            pltpu.VMEM_SHARED((ACCR, NH), f32),
            pltpu.VMEM((CHUNK + 8,), jnp.int32),
            pltpu.VMEM((CHUNK + 8,), jnp.int32),
            pltpu.VMEM((CAPC,), jnp.int32),
            pltpu.VMEM((CAPC,), jnp.int32),
            pltpu.VMEM((G,), jnp.int32),
            pltpu.VMEM((G,), jnp.int32),
            pltpu.VMEM((G,), jnp.int32),
            pltpu.VMEM((G, HID), f32),
            pltpu.VMEM((G, 2 * NH), f32),
            pltpu.VMEM((G, 2 * NH), f32),
            pltpu.VMEM((G, NH), f32),
            pltpu.SemaphoreType.DMA,
        ],
    )(src, dst, xw, asad, z64, z8)


def _fin_body(acc_ref, den_ref, xw_ref, asad_ref, res_ref, bias_ref, o_ref):
    rep = jnp.kron(jnp.eye(NH, dtype=jnp.float32),
                   jnp.ones((1, CH), jnp.float32))
    a = asad_ref[:, :NH] + asad_ref[:, NH:]
    a = jnp.where(a >= 0, a, 0.2 * a)
    exs = jnp.exp(a)
    den = (den_ref[...] + exs) @ rep
    acc = acc_ref[...] + xw_ref[...] * (exs @ rep)
    o_ref[...] = acc / den + res_ref[...] + bias_ref[...]


def _finalize(acc, den, xw, asad, res, bias):
    blk = lambda c: pl.BlockSpec((NBLK, c), lambda i: (i, 0))
    full = lambda shape: pl.BlockSpec(shape, lambda i: tuple(0 for _ in shape))
    return pl.pallas_call(
        _fin_body,
        grid=(N // NBLK,),
        in_specs=[blk(HID), blk(NH), blk(HID), blk(2 * NH), blk(HID),
                  full((HID,))],
        out_specs=pl.BlockSpec((NBLK, HID), lambda i: (i, 0)),
        out_shape=jax.ShapeDtypeStruct((N, HID), jnp.float32),
    )(acc, den, xw, asad, res, bias)


def _gat_layer(x, src, dst, z64, z8, lg, lb, w, a_src, a_dst, res_w, bias):
    xw, asad, res = _gat_pre(x, lg, lb, w, a_src, a_dst, res_w)
    acc, den = _edge_call(src, dst, xw, asad, z64, z8)
    return _finalize(acc[:N], den[:N], xw, asad, res, bias)


def kernel(u_proj, ps_proj, pf_proj, pb_proj, edge_index, fc_ln_g, fc_ln_b,
           fc_W, fc_b, ln_p_g, ln_p_b, W_p, att_src_p, att_dst_p, res_p,
           bias_p, ln_u_g, ln_u_b, W_u, att_src_u, att_dst_u, res_u, bias_u):
    src = edge_index[0].astype(jnp.int32)
    dst = edge_index[1].astype(jnp.int32)
    z64 = jnp.zeros((OUTR, HID), jnp.float32)
    z8 = jnp.zeros((OUTR, NH), jnp.float32)
    p = _fc_stage(ps_proj, pf_proj, pb_proj, fc_ln_g, fc_ln_b, fc_W, fc_b)
    u1 = _gat_layer(p, src, dst, z64, z8, ln_p_g, ln_p_b, W_p, att_src_p,
                    att_dst_p, res_p, bias_p)
    out = _gat_layer(u1, src, dst, z64, z8, ln_u_g, ln_u_b, W_u, att_src_u,
                     att_dst_u, res_u, bias_u)
    return out.reshape(u_proj.shape)
